# Initial kernel scaffold; baseline (speedup 1.0000x reference)
import jax
import jax.numpy as jnp
from jax.experimental import pallas as pl


def kernel(movie_ids, genre_ids, movie_table, genre_table, W1, b1, W2, b2):
    raise NotImplementedError("write your pallas kernel here")


# speedup vs baseline: 4.7531x; 4.7531x over previous
import functools

import jax
import jax.numpy as jnp
from jax import lax
from jax.experimental import pallas as pl
from jax.experimental.pallas import tpu as pltpu
from jax.experimental.pallas import tpu_sc as plsc

B = 4096
L = 20
D = 32
H1 = 256
OUT = 64

NW = 32
BPW = B // NW
GPW = BPW * L
GCH = 128
NCH = GPW // GCH


def _sc_feat(movie_ids, genre_ids_flat, movie_table, genre_table):
    mesh = plsc.VectorSubcoreMesh(core_axis_name="c", subcore_axis_name="s")

    @functools.partial(
        pl.kernel,
        mesh=mesh,
        out_type=jax.ShapeDtypeStruct((B, 2 * D), jnp.float32),
        scratch_types=[
            pltpu.VMEM((BPW,), jnp.int32),
            pltpu.VMEM((GPW,), jnp.int32),
            pltpu.VMEM((BPW, D), jnp.float32),
            pltpu.VMEM((GPW, D), jnp.float32),
            pltpu.VMEM((BPW, 2 * D), jnp.float32),
            pltpu.SemaphoreType.DMA,
        ],
        compiler_params=pltpu.CompilerParams(use_tc_tiling_on_sc=False),
    )
    def feat_kernel(mids_hbm, gids_hbm, mtab_hbm, gtab_hbm, out_hbm,
                    midx, gidx, mrows, grows, feat, sem):
        wid = lax.axis_index("s") * 2 + lax.axis_index("c")
        base = wid * BPW
        pltpu.sync_copy(mids_hbm.at[pl.ds(base, BPW)], midx)
        pltpu.sync_copy(gids_hbm.at[pl.ds(base * L, GPW)], gidx)
        copies = [pltpu.async_copy(mtab_hbm.at[midx], mrows, sem)]
        for j in range(NCH):
            copies.append(
                pltpu.async_copy(
                    gtab_hbm.at[gidx.at[pl.ds(j * GCH, GCH)]],
                    grows.at[pl.ds(j * GCH, GCH)],
                    sem,
                )
            )
        for cp in copies:
            cp.wait()

        def body(b, carry):
            for j in range(D // 16):
                feat[b, pl.ds(j * 16, 16)] = mrows[b, pl.ds(j * 16, 16)]
            for j in range(D // 16):
                acc = grows[b * L, pl.ds(j * 16, 16)]
                for l in range(1, L):
                    acc = acc + grows[b * L + l, pl.ds(j * 16, 16)]
                feat[b, pl.ds(D + j * 16, 16)] = acc
            return carry

        lax.fori_loop(0, BPW, body, 0)
        pltpu.sync_copy(feat, out_hbm.at[pl.ds(base, BPW)])

    return feat_kernel(movie_ids, genre_ids_flat, movie_table, genre_table)


def _tc_mlp(feat_raw, genre_ids, row0, W1, b1, W2, b2):
    BLK = 512

    def mlp_body(feat_ref, gid_ref, row0_ref, W1_ref, b1_ref, W2_ref, b2_ref,
                 out_ref):
        feat = feat_ref[...]
        gids = gid_ref[...]
        c0 = jnp.sum((gids == 0).astype(jnp.float32), axis=1, keepdims=True)
        denom = jnp.maximum(jnp.float32(L) - c0, 1.0)
        m = feat[:, :D]
        g = (feat[:, D:] - c0 * row0_ref[...]) / denom
        h = jnp.maximum(
            jnp.dot(m, W1_ref[:D, :], preferred_element_type=jnp.float32)
            + jnp.dot(g, W1_ref[D:, :], preferred_element_type=jnp.float32)
            + b1_ref[...],
            0.0,
        )
        out_ref[...] = (
            jnp.dot(h, W2_ref[...], preferred_element_type=jnp.float32)
            + b2_ref[...]
        )

    return pl.pallas_call(
        mlp_body,
        grid=(B // BLK,),
        in_specs=[
            pl.BlockSpec((BLK, 2 * D), lambda i: (i, 0)),
            pl.BlockSpec((BLK, L), lambda i: (i, 0)),
            pl.BlockSpec((1, D), lambda i: (0, 0)),
            pl.BlockSpec((2 * D, H1), lambda i: (0, 0)),
            pl.BlockSpec((1, H1), lambda i: (0, 0)),
            pl.BlockSpec((H1, OUT), lambda i: (0, 0)),
            pl.BlockSpec((1, OUT), lambda i: (0, 0)),
        ],
        out_specs=pl.BlockSpec((BLK, OUT), lambda i: (i, 0)),
        out_shape=jax.ShapeDtypeStruct((B, OUT), jnp.float32),
        compiler_params=pltpu.CompilerParams(
            dimension_semantics=("parallel",),
        ),
    )(feat_raw, genre_ids, row0, W1, b1, W2, b2)


def kernel(movie_ids, genre_ids, movie_table, genre_table, W1, b1, W2, b2):
    mids = movie_ids.astype(jnp.int32)
    gids = genre_ids.astype(jnp.int32)
    feat_raw = _sc_feat(mids, gids.reshape(B * L), movie_table, genre_table)
    row0 = genre_table[0:1, :]
    return _tc_mlp(feat_raw, gids, row0,
                   W1, b1.reshape(1, H1), W2, b2.reshape(1, OUT))


# speedup vs baseline: 5.3488x; 1.1253x over previous
import functools

import jax
import jax.numpy as jnp
from jax import lax
from jax.experimental import pallas as pl
from jax.experimental.pallas import tpu as pltpu
from jax.experimental.pallas import tpu_sc as plsc

B = 4096
L = 20
D = 32
H1 = 256
OUT = 64

NW = 32
BPW = B // NW
GPW = BPW * L
GCH = 128
NCH = GPW // GCH

_MESH = plsc.VectorSubcoreMesh(core_axis_name="c", subcore_axis_name="s")


def _sc_genre_sum(genre_ids_flat, genre_table):
    @functools.partial(
        pl.kernel,
        mesh=_MESH,
        out_type=jax.ShapeDtypeStruct((B, D), jnp.float32),
        scratch_types=[
            pltpu.VMEM((GPW,), jnp.int32),
            pltpu.VMEM((GPW, D), jnp.float32),
            pltpu.VMEM((BPW, D), jnp.float32),
            pltpu.SemaphoreType.DMA,
        ],
        compiler_params=pltpu.CompilerParams(use_tc_tiling_on_sc=False),
    )
    def genre_kernel(gids_hbm, gtab_hbm, out_hbm, gidx, grows, gsum, sem):
        wid = lax.axis_index("s") * 2 + lax.axis_index("c")
        base = wid * BPW
        pltpu.sync_copy(gids_hbm.at[pl.ds(base * L, GPW)], gidx)
        copies = []
        for j in range(NCH):
            copies.append(
                pltpu.async_copy(
                    gtab_hbm.at[gidx.at[pl.ds(j * GCH, GCH)]],
                    grows.at[pl.ds(j * GCH, GCH)],
                    sem,
                )
            )
        for cp in copies:
            cp.wait()

        def body(b, carry):
            for j in range(D // 16):
                acc = grows[b * L, pl.ds(j * 16, 16)]
                for l in range(1, L):
                    acc = acc + grows[b * L + l, pl.ds(j * 16, 16)]
                gsum[b, pl.ds(j * 16, 16)] = acc
            return carry

        lax.fori_loop(0, BPW, body, 0)
        pltpu.sync_copy(gsum, out_hbm.at[pl.ds(base, BPW)])

    return genre_kernel(genre_ids_flat, genre_table)


def _sc_movie_rows(movie_ids, movie_table):
    @functools.partial(
        pl.kernel,
        mesh=_MESH,
        out_type=jax.ShapeDtypeStruct((B, D), jnp.float32),
        scratch_types=[
            pltpu.VMEM((BPW,), jnp.int32),
            pltpu.VMEM((BPW, D), jnp.float32),
            pltpu.SemaphoreType.DMA,
        ],
        compiler_params=pltpu.CompilerParams(use_tc_tiling_on_sc=False),
    )
    def movie_kernel(mids_hbm, mtab_hbm, out_hbm, midx, mrows, sem):
        wid = lax.axis_index("s") * 2 + lax.axis_index("c")
        base = wid * BPW
        pltpu.sync_copy(mids_hbm.at[pl.ds(base, BPW)], midx)
        pltpu.async_copy(mtab_hbm.at[midx], mrows, sem).wait()
        pltpu.sync_copy(mrows, out_hbm.at[pl.ds(base, BPW)])

    return movie_kernel(movie_ids, movie_table)


def _tc_mlp(m, gsum, genre_ids, row0, W1, b1, W2, b2):
    BLK = 512

    def mlp_body(m_ref, g_ref, gid_ref, row0_ref, W1_ref, b1_ref, W2_ref,
                 b2_ref, out_ref):
        gids = gid_ref[...]
        c0 = jnp.sum((gids == 0).astype(jnp.float32), axis=1, keepdims=True)
        denom = jnp.maximum(jnp.float32(L) - c0, 1.0)
        g = (g_ref[...] - c0 * row0_ref[...]) / denom
        h = jnp.maximum(
            jnp.dot(m_ref[...], W1_ref[:D, :], preferred_element_type=jnp.float32)
            + jnp.dot(g, W1_ref[D:, :], preferred_element_type=jnp.float32)
            + b1_ref[...],
            0.0,
        )
        out_ref[...] = (
            jnp.dot(h, W2_ref[...], preferred_element_type=jnp.float32)
            + b2_ref[...]
        )

    return pl.pallas_call(
        mlp_body,
        grid=(B // BLK,),
        in_specs=[
            pl.BlockSpec((BLK, D), lambda i: (i, 0)),
            pl.BlockSpec((BLK, D), lambda i: (i, 0)),
            pl.BlockSpec((BLK, L), lambda i: (i, 0)),
            pl.BlockSpec((1, D), lambda i: (0, 0)),
            pl.BlockSpec((2 * D, H1), lambda i: (0, 0)),
            pl.BlockSpec((1, H1), lambda i: (0, 0)),
            pl.BlockSpec((H1, OUT), lambda i: (0, 0)),
            pl.BlockSpec((1, OUT), lambda i: (0, 0)),
        ],
        out_specs=pl.BlockSpec((BLK, OUT), lambda i: (i, 0)),
        out_shape=jax.ShapeDtypeStruct((B, OUT), jnp.float32),
        compiler_params=pltpu.CompilerParams(
            dimension_semantics=("parallel",),
        ),
    )(m, gsum, genre_ids, row0, W1, b1, W2, b2)


def kernel(movie_ids, genre_ids, movie_table, genre_table, W1, b1, W2, b2):
    mids = movie_ids.astype(jnp.int32)
    gids = genre_ids.astype(jnp.int32)
    gsum = _sc_genre_sum(gids.reshape(B * L), genre_table)
    m = _sc_movie_rows(mids, movie_table)
    row0 = genre_table[0:1, :]
    return _tc_mlp(m, gsum, gids, row0,
                   W1, b1.reshape(1, H1), W2, b2.reshape(1, OUT))
